# Initial kernel scaffold; baseline (speedup 1.0000x reference)
import jax
import jax.numpy as jnp
from jax.experimental import pallas as pl


def kernel(n_words, bigram, start, end):
    raise NotImplementedError("write your pallas kernel here")


# speedup vs baseline: 57.8154x; 57.8154x over previous
import functools

import jax
import jax.numpy as jnp
from jax import lax
from jax.experimental import pallas as pl
from jax.experimental.pallas import tpu as pltpu
from jax.experimental.pallas import tpu_sc as plsc

CHAIN = 1280
NTH = 10
NWORDS = 512
NC, NS = 2, 16
NWK = NC * NS
ROWS_PER = CHAIN // NWK
IDX_PER_ROW = NWORDS + 16
IDX_PER_TILE = ROWS_PER * IDX_PER_ROW
CHUNK = 128
NCHUNK = IDX_PER_TILE // CHUNK
DEPTH = 6
NSEG = IDX_PER_ROW // 16

_mesh = plsc.VectorSubcoreMesh(core_axis_name="c", subcore_axis_name="s")
_params = pltpu.CompilerParams(needs_layout_passes=False)


@functools.partial(
    pl.kernel,
    out_type=jax.ShapeDtypeStruct((CHAIN,), jnp.float32),
    mesh=_mesh,
    scratch_types=[
        pltpu.VMEM((IDX_PER_TILE,), jnp.int32),
        pltpu.VMEM((48 * IDX_PER_ROW,), jnp.float32),
        pltpu.VMEM((48,), jnp.float32),
        pltpu.SemaphoreType.DMA,
    ],
    compiler_params=_params,
)
def _score(table_hbm, idx_hbm, w_hbm, idx_v, val_v, w_v, sem):
    wid = lax.axis_index("s") * NC + lax.axis_index("c")
    pltpu.sync_copy(
        idx_hbm.at[pl.ds(pl.multiple_of(wid * IDX_PER_TILE, 8), IDX_PER_TILE)],
        idx_v)

    def _chunk_slice(ref, j):
        return ref.at[pl.ds(pl.multiple_of(j * CHUNK, 8), CHUNK)]

    def _issue(j):
        pltpu.async_copy(
            table_hbm.at[_chunk_slice(idx_v, j)],
            _chunk_slice(val_v, j),
            sem,
        )

    def _drain(j):
        pltpu.make_async_copy(
            table_hbm.at[pl.ds(0, CHUNK)],
            _chunk_slice(val_v, j),
            sem,
        ).wait()

    for j in range(DEPTH):
        _issue(j)

    def _body(j, carry):
        _drain(j)
        _issue(j + DEPTH)
        return carry

    lax.fori_loop(0, NCHUNK - DEPTH, _body, 0)
    for j in range(NCHUNK - DEPTH, NCHUNK):
        _drain(j)

    lane = jnp.arange(16, dtype=jnp.int32)
    for g in range(3):
        base = (g * 16 + lane) * IDX_PER_ROW

        def _ksum(k8, acc):
            for d in range(8):
                acc = acc + plsc.load_gather(val_v, [base + (k8 * 8 + d)])
            return acc

        acc = lax.fori_loop(0, IDX_PER_ROW // 8, _ksum,
                            jnp.zeros((16,), jnp.float32))
        w_v[pl.ds(g * 16, 16)] = acc
    pltpu.sync_copy(w_v.at[pl.ds(0, ROWS_PER)],
                    w_hbm.at[pl.ds(wid * ROWS_PER, ROWS_PER)])


@functools.partial(
    pl.kernel,
    out_type=jax.ShapeDtypeStruct((CHAIN // NTH, NWORDS), jnp.int32),
    mesh=_mesh,
    scratch_types=[
        pltpu.VMEM((CHAIN,), jnp.float32),
        pltpu.VMEM((CHAIN,), jnp.float32),
        pltpu.VMEM((CHAIN // NTH,), jnp.int32),
        pltpu.VMEM((CHAIN // NTH, NWORDS), jnp.int32),
        pltpu.SemaphoreType.DMA,
    ],
    compiler_params=_params,
)
def _chain(w_hbm, lu_hbm, perm_hbm, out_hbm, w_v, lu_v, sel_v, rows_v, sem):
    wid = lax.axis_index("s") * NC + lax.axis_index("c")

    @pl.when(wid == 0)
    def _():
        pltpu.sync_copy(w_hbm, w_v)
        pltpu.sync_copy(lu_hbm, lu_v)
        lane = jnp.arange(16, dtype=jnp.int32)

        def _block(b, carry, t0):
            w_last, src, sel_acc = carry
            w16 = w_v[pl.ds(pl.multiple_of(b * 16, 16), 16)]
            lu16 = lu_v[pl.ds(pl.multiple_of(b * 16, 16), 16)]
            for t in range(t0, 16):
                i = b * 16 + t
                acc = (w16[t] - w_last) > lu16[t]
                src = jnp.where(acc, i, src)
                w_last = jnp.where(acc, w16[t], w_last)
                tgt = jnp.where(i % NTH == NTH - 1,
                                (i // NTH) % 16, jnp.int32(-1))
                sel_acc = jnp.where(lane == tgt, src, sel_acc)

            @pl.when(b % NTH == NTH - 1)
            def _():
                sel_v[pl.ds(pl.multiple_of((b // NTH) * 16, 16), 16)] = sel_acc

            return (w_last, src, sel_acc)

        w0 = w_v[pl.ds(0, 16)][0]
        carry = _block(0, (w0, jnp.int32(0), jnp.zeros((16,), jnp.int32)), 1)
        lax.fori_loop(1, CHAIN // 16, lambda b, c: _block(b, c, 0), carry)
        pltpu.async_copy(perm_hbm.at[sel_v], rows_v, sem).wait()
        pltpu.sync_copy(rows_v, out_hbm)


def kernel(n_words, bigram, start, end):
    del n_words
    nw = bigram.shape[0]
    key = jax.random.key(42)
    kg, ku = jax.random.split(key)
    rand = jax.random.gumbel(kg, (CHAIN, nw), dtype=jnp.float32)
    perm = jnp.argsort(rand, axis=1)
    u = jax.random.uniform(ku, (CHAIN,), dtype=jnp.float32)
    lu = jnp.log(u)

    zidx = nw * nw + 2 * nw
    idx = jnp.concatenate(
        [
            perm[:, :-1] * nw + perm[:, 1:],
            nw * nw + perm[:, :1],
            nw * nw + nw + perm[:, -1:],
            jnp.full((CHAIN, IDX_PER_ROW - NWORDS - 1), zidx, jnp.int32),
        ],
        axis=1,
    ).reshape(NWK * IDX_PER_TILE)
    table = jnp.concatenate(
        [bigram.reshape(-1), start, end, jnp.zeros((8,), jnp.float32)]
    )

    w = _score(table, idx)
    return _chain(w, lu, perm)


# speedup vs baseline: 57.9719x; 1.0027x over previous
import functools

import jax
import jax.numpy as jnp
from jax import lax
from jax.experimental import pallas as pl
from jax.experimental.pallas import tpu as pltpu
from jax.experimental.pallas import tpu_sc as plsc

CHAIN = 1280
NTH = 10
NWORDS = 512
NC, NS = 2, 16
NWK = NC * NS
ROWS_PER = CHAIN // NWK
IDX_PER_ROW = NWORDS + 16
IDX_PER_TILE = ROWS_PER * IDX_PER_ROW
CHUNK = 128
NCHUNK = IDX_PER_TILE // CHUNK
DEPTH = 6
NSEG = IDX_PER_ROW // 16

_mesh = plsc.VectorSubcoreMesh(core_axis_name="c", subcore_axis_name="s")
_params = pltpu.CompilerParams(needs_layout_passes=False)


@functools.partial(
    pl.kernel,
    out_type=jax.ShapeDtypeStruct((CHAIN,), jnp.float32),
    mesh=_mesh,
    scratch_types=[
        pltpu.VMEM((IDX_PER_TILE,), jnp.int32),
        pltpu.VMEM((48 * IDX_PER_ROW,), jnp.float32),
        pltpu.VMEM((48,), jnp.float32),
        pltpu.SemaphoreType.DMA,
    ],
    compiler_params=_params,
)
def _score(table_hbm, idx_hbm, w_hbm, idx_v, val_v, w_v, sem):
    wid = lax.axis_index("s") * NC + lax.axis_index("c")
    pltpu.sync_copy(
        idx_hbm.at[pl.ds(pl.multiple_of(wid * IDX_PER_TILE, 8), IDX_PER_TILE)],
        idx_v)

    pltpu.async_copy(
        table_hbm.at[idx_v],
        val_v.at[pl.ds(0, IDX_PER_TILE)],
        sem,
    ).wait()

    lane = jnp.arange(16, dtype=jnp.int32)
    for g in range(3):
        base = (g * 16 + lane) * IDX_PER_ROW

        def _ksum(k8, acc):
            for d in range(8):
                acc = acc + plsc.load_gather(val_v, [base + (k8 * 8 + d)])
            return acc

        acc = lax.fori_loop(0, IDX_PER_ROW // 8, _ksum,
                            jnp.zeros((16,), jnp.float32))
        w_v[pl.ds(g * 16, 16)] = acc
    pltpu.sync_copy(w_v.at[pl.ds(0, ROWS_PER)],
                    w_hbm.at[pl.ds(wid * ROWS_PER, ROWS_PER)])


@functools.partial(
    pl.kernel,
    out_type=jax.ShapeDtypeStruct((CHAIN // NTH, NWORDS), jnp.int32),
    mesh=_mesh,
    scratch_types=[
        pltpu.VMEM((CHAIN,), jnp.float32),
        pltpu.VMEM((CHAIN,), jnp.float32),
        pltpu.VMEM((CHAIN // NTH,), jnp.int32),
        pltpu.VMEM((CHAIN // NTH, NWORDS), jnp.int32),
        pltpu.SemaphoreType.DMA,
    ],
    compiler_params=_params,
)
def _chain(w_hbm, lu_hbm, perm_hbm, out_hbm, w_v, lu_v, sel_v, rows_v, sem):
    wid = lax.axis_index("s") * NC + lax.axis_index("c")

    @pl.when(wid == 0)
    def _():
        pltpu.sync_copy(w_hbm, w_v)
        pltpu.sync_copy(lu_hbm, lu_v)
        lane = jnp.arange(16, dtype=jnp.int32)

        def _block(b, carry, t0):
            w_last, src, sel_acc = carry
            w16 = w_v[pl.ds(pl.multiple_of(b * 16, 16), 16)]
            lu16 = lu_v[pl.ds(pl.multiple_of(b * 16, 16), 16)]
            for t in range(t0, 16):
                i = b * 16 + t
                acc = (w16[t] - w_last) > lu16[t]
                src = jnp.where(acc, i, src)
                w_last = jnp.where(acc, w16[t], w_last)
                tgt = jnp.where(i % NTH == NTH - 1,
                                (i // NTH) % 16, jnp.int32(-1))
                sel_acc = jnp.where(lane == tgt, src, sel_acc)

            @pl.when(b % NTH == NTH - 1)
            def _():
                sel_v[pl.ds(pl.multiple_of((b // NTH) * 16, 16), 16)] = sel_acc

            return (w_last, src, sel_acc)

        w0 = w_v[pl.ds(0, 16)][0]
        carry = _block(0, (w0, jnp.int32(0), jnp.zeros((16,), jnp.int32)), 1)
        lax.fori_loop(1, CHAIN // 16, lambda b, c: _block(b, c, 0), carry)
        pltpu.async_copy(perm_hbm.at[sel_v], rows_v, sem).wait()
        pltpu.sync_copy(rows_v, out_hbm)


def kernel(n_words, bigram, start, end):
    del n_words
    nw = bigram.shape[0]
    key = jax.random.key(42)
    kg, ku = jax.random.split(key)
    rand = jax.random.gumbel(kg, (CHAIN, nw), dtype=jnp.float32)
    perm = jnp.argsort(rand, axis=1)
    u = jax.random.uniform(ku, (CHAIN,), dtype=jnp.float32)
    lu = jnp.log(u)

    zidx = nw * nw + 2 * nw
    idx = jnp.concatenate(
        [
            perm[:, :-1] * nw + perm[:, 1:],
            nw * nw + perm[:, :1],
            nw * nw + nw + perm[:, -1:],
            jnp.full((CHAIN, IDX_PER_ROW - NWORDS - 1), zidx, jnp.int32),
        ],
        axis=1,
    ).reshape(NWK * IDX_PER_TILE)
    table = jnp.concatenate(
        [bigram.reshape(-1), start, end, jnp.zeros((8,), jnp.float32)]
    )

    w = _score(table, idx)
    return _chain(w, lu, perm)
